# baseline (speedup 1.0000x reference)
import functools

import jax
import jax.numpy as jnp
from jax import lax
from jax.experimental import pallas as pl
from jax.experimental.pallas import tpu as pltpu
from jax.experimental.pallas import tpu_sc as plsc

LANES = 16


def _sum_splat(v):
    idx = lax.iota(jnp.int32, 16)
    for k in (8, 4, 2, 1):
        v = v + v.at[jnp.bitwise_xor(idx, jnp.int32(k))].get(
            mode="promise_in_bounds")
    return v


def _rsqrt_newton(x):
    i = lax.bitcast_convert_type(x, jnp.int32)
    i = jnp.int32(0x5F3759DF) - (i >> 1)
    y = lax.bitcast_convert_type(i, jnp.float32)
    half = x * jnp.float32(0.5)
    for _ in range(3):
        y = y * (jnp.float32(1.5) - half * y * y)
    return y


def _make_sc_kernel(B, L, V, D):
    info = plsc.get_sparse_core_info()
    NC, NS = info.num_cores, info.num_subcores
    NW = NC * NS
    assert B % NW == 0
    bpw = B // NW
    rows_per_chunk = 2
    assert bpw % rows_per_chunk == 0
    chunks = bpw // rows_per_chunk
    assert chunks % 2 == 0
    clen = rows_per_chunk * L
    ngrp = D // LANES

    mesh = plsc.VectorSubcoreMesh(core_axis_name="c", subcore_axis_name="s")

    @functools.partial(
        pl.kernel,
        mesh=mesh,
        out_type=jax.ShapeDtypeStruct((B, D), jnp.float32),
        compiler_params=pltpu.CompilerParams(use_tc_tiling_on_sc=False),
        scratch_types=[
            pltpu.VMEM((chunks, clen), jnp.int32),
            [pltpu.VMEM((clen, D), jnp.float32) for _ in range(4)],
            pltpu.VMEM((bpw, D), jnp.float32),
            [pltpu.SemaphoreType.DMA for _ in range(4)],
        ],
    )
    def sc_kernel(idx_hbm, table_hbm, out_hbm, idx_v, bufs, out_v, sems):
        wid = lax.axis_index("s") * NC + lax.axis_index("c")
        pltpu.sync_copy(idx_hbm.at[pl.ds(wid * chunks, chunks)], idx_v)

        def compute(buf, out_base):
            for r in range(rows_per_chunk):
                accs = [buf[r * L, pl.ds(LANES * c, LANES)]
                        for c in range(ngrp)]
                for j in range(1, L):
                    for c in range(ngrp):
                        accs[c] = accs[c] + buf[r * L + j,
                                                pl.ds(LANES * c, LANES)]
                sq = accs[0] * accs[0]
                for c in range(1, ngrp):
                    sq = sq + accs[c] * accs[c]
                n2 = jnp.maximum(_sum_splat(sq), jnp.float32(1e-24))
                inv = _rsqrt_newton(n2)
                for c in range(ngrp):
                    out_v[out_base + r, pl.ds(LANES * c, LANES)] = accs[c] * inv

        def fire(g, i):
            pltpu.async_copy(table_hbm.at[idx_v.at[g]], bufs[i], sems[i])

        def wait(i):
            pltpu.make_async_copy(table_hbm.at[idx_v.at[0]], bufs[i],
                                  sems[i]).wait()

        for i in range(4):
            fire(i, i)

        def quad_body(h, _):
            g0 = 4 * h
            for i in range(4):
                wait(i)
                compute(bufs[i], (g0 + i) * rows_per_chunk)
                fire(jnp.minimum(g0 + 4 + i, chunks - 1), i)
            return 0

        lax.fori_loop(0, chunks // 4, quad_body, 0)
        for i in range(4):
            wait(i)
        pltpu.sync_copy(out_v, out_hbm.at[pl.ds(wid * bpw, bpw)])

    return sc_kernel


def kernel(text_idxs, text_len, embedding_table):
    del text_len
    B, L = text_idxs.shape
    V, D = embedding_table.shape
    rows_per_chunk = 2
    clen = rows_per_chunk * L
    idx2d = text_idxs.astype(jnp.int32).reshape(B * L // clen, clen)
    sc = _make_sc_kernel(B, L, V, D)
    return sc(idx2d, embedding_table)


# speedup vs baseline: 1.1210x; 1.1210x over previous
import functools

import jax
import jax.numpy as jnp
from jax import lax
from jax.experimental import pallas as pl
from jax.experimental.pallas import tpu as pltpu
from jax.experimental.pallas import tpu_sc as plsc

LANES = 16


def _sum_splat(v):
    idx = lax.iota(jnp.int32, 16)
    for k in (8, 4, 2, 1):
        v = v + v.at[jnp.bitwise_xor(idx, jnp.int32(k))].get(
            mode="promise_in_bounds")
    return v


def _rsqrt_newton(x):
    i = lax.bitcast_convert_type(x, jnp.int32)
    i = jnp.int32(0x5F3759DF) - (i >> 1)
    y = lax.bitcast_convert_type(i, jnp.float32)
    half = x * jnp.float32(0.5)
    for _ in range(3):
        y = y * (jnp.float32(1.5) - half * y * y)
    return y


def _make_sc_kernel(B, L, V, D):
    info = plsc.get_sparse_core_info()
    NC, NS = info.num_cores, info.num_subcores
    NW = NC * NS
    assert B % NW == 0
    bpw = B // NW
    rows_per_chunk = 2
    assert bpw % rows_per_chunk == 0
    chunks = bpw // rows_per_chunk
    assert chunks % 2 == 0
    clen = rows_per_chunk * L
    ngrp = D // LANES

    mesh = plsc.VectorSubcoreMesh(core_axis_name="c", subcore_axis_name="s")

    @functools.partial(
        pl.kernel,
        mesh=mesh,
        out_type=jax.ShapeDtypeStruct((B, D), jnp.float32),
        compiler_params=pltpu.CompilerParams(use_tc_tiling_on_sc=False),
        scratch_types=[
            pltpu.VMEM((chunks, clen), jnp.int32),
            pltpu.VMEM((clen, D), jnp.float32),
            pltpu.VMEM((clen, D), jnp.float32),
            pltpu.VMEM((bpw, D), jnp.float32),
            pltpu.SemaphoreType.DMA,
            pltpu.SemaphoreType.DMA,
        ],
    )
    def sc_kernel(idx_hbm, table_hbm, out_hbm, idx_v, rows0, rows1, out_v,
                  sem0, sem1):
        wid = lax.axis_index("s") * NC + lax.axis_index("c")
        pltpu.sync_copy(idx_hbm.at[pl.ds(wid * chunks, chunks)], idx_v)

        def compute(buf, out_base):
            for r in range(rows_per_chunk):
                accs = [buf[r * L, pl.ds(LANES * c, LANES)]
                        for c in range(ngrp)]
                for j in range(1, L):
                    for c in range(ngrp):
                        accs[c] = accs[c] + buf[r * L + j,
                                                pl.ds(LANES * c, LANES)]
                sq = accs[0] * accs[0]
                for c in range(1, ngrp):
                    sq = sq + accs[c] * accs[c]
                n2 = jnp.maximum(_sum_splat(sq), jnp.float32(1e-24))
                inv = _rsqrt_newton(n2)
                for c in range(ngrp):
                    out_v[out_base + r, pl.ds(LANES * c, LANES)] = accs[c] * inv

        pltpu.async_copy(table_hbm.at[idx_v.at[0]], rows0, sem0)

        def pair_body(h, _):
            g0 = 2 * h
            pltpu.async_copy(table_hbm.at[idx_v.at[g0 + 1]], rows1, sem1)
            pltpu.make_async_copy(table_hbm.at[idx_v.at[g0]], rows0,
                                  sem0).wait()
            compute(rows0, g0 * rows_per_chunk)
            gnext = jnp.minimum(g0 + 2, chunks - 1)
            pltpu.async_copy(table_hbm.at[idx_v.at[gnext]], rows0, sem0)
            pltpu.make_async_copy(table_hbm.at[idx_v.at[g0 + 1]], rows1,
                                  sem1).wait()
            compute(rows1, (g0 + 1) * rows_per_chunk)
            return 0

        lax.fori_loop(0, chunks // 2, pair_body, 0)
        pltpu.make_async_copy(table_hbm.at[idx_v.at[0]], rows0, sem0).wait()
        pltpu.sync_copy(out_v, out_hbm.at[pl.ds(wid * bpw, bpw)])

    return sc_kernel


def kernel(text_idxs, text_len, embedding_table):
    del text_len
    B, L = text_idxs.shape
    V, D = embedding_table.shape
    rows_per_chunk = 2
    clen = rows_per_chunk * L
    idx2d = text_idxs.astype(jnp.int32).reshape(B * L // clen, clen)
    sc = _make_sc_kernel(B, L, V, D)
    return sc(idx2d, embedding_table)
